# baseline (speedup 1.0000x reference)
import functools

import jax
import jax.numpy as jnp
from jax import lax
from jax.experimental import pallas as pl
from jax.experimental.pallas import tpu as pltpu
from jax.experimental.pallas import tpu_sc as plsc

B = 4096
L = 200
LPAD = 208
HALF = LPAD // 2
E = 64
OUT = 10

NC = 2
NS = 16
NW = NC * NS
ROWS_PER_W = B // NW

NB = 4


def _pool_body(x2_hbm, table_hbm, out_hbm, idx_v, rows_v, p_buf, *sems):
    wid = lax.axis_index("s") * NC + lax.axis_index("c")
    base = wid * ROWS_PER_W
    pltpu.sync_copy(x2_hbm.at[pl.ds(base * 2, 2 * ROWS_PER_W)], idx_v)

    neg = jnp.full((32,), -jnp.inf, dtype=jnp.bfloat16)

    def issue(slot, i):
        for h in range(2):
            pltpu.async_copy(
                table_hbm.at[idx_v.at[2 * i + h]],
                rows_v.at[pl.ds((2 * slot + h) * HALF, HALF)], sems[slot])

    def drain(slot):
        for h in range(2):
            pltpu.make_async_copy(
                table_hbm.at[idx_v.at[h]],
                rows_v.at[pl.ds((2 * slot + h) * HALF, HALF)],
                sems[slot]).wait()

    for s in range(NB):
        issue(s, s)

    def blk_body(g, carry):
        for s in range(NB):
            i = g * NB + s
            drain(s)

            def seq_body(j, acc):
                a0, a1 = acc
                for jj in range(8):
                    r = rows_v.at[2 * s * HALF + j * 8 + jj]
                    a0 = jnp.maximum(a0, r[pl.ds(0, 32)])
                    a1 = jnp.maximum(a1, r[pl.ds(32, 32)])
                return (a0, a1)

            acc = lax.fori_loop(0, 2 * HALF // 8, seq_body, (neg, neg))
            p_buf[i, pl.ds(0, 32)] = acc[0]
            p_buf[i, pl.ds(32, 32)] = acc[1]

            nxt = i + NB

            @pl.when(nxt < ROWS_PER_W)
            def _():
                issue(s, nxt)
        return carry

    lax.fori_loop(0, ROWS_PER_W // NB, blk_body, 0)
    pltpu.sync_copy(p_buf, out_hbm.at[pl.ds(base, ROWS_PER_W)])


_pool = functools.partial(
    pl.kernel,
    mesh=plsc.VectorSubcoreMesh(
        core_axis_name="c", subcore_axis_name="s",
        num_cores=NC, num_subcores=NS,
    ),
    out_type=jax.ShapeDtypeStruct((B, E), jnp.bfloat16),
    scratch_types=[
        pltpu.VMEM((2 * ROWS_PER_W, HALF), jnp.int32),
        pltpu.VMEM((2 * NB * HALF, E), jnp.bfloat16),
        pltpu.VMEM((ROWS_PER_W, E), jnp.bfloat16),
    ] + [pltpu.SemaphoreType.DMA] * NB,
    compiler_params=pltpu.CompilerParams(use_tc_tiling_on_sc=False),
)(_pool_body)


def _linear_body(p_ref, w_ref, b_ref, o_ref):
    h = jnp.maximum(p_ref[...].astype(jnp.float32), 0.0)
    o_ref[...] = (
        jnp.dot(h, w_ref[...], preferred_element_type=jnp.float32) + b_ref[...]
    )


def kernel(x, emb_table, fc_w, fc_b):
    x = x.astype(jnp.int32)
    x_pad = jnp.concatenate([x, x[:, L - (LPAD - L):]], axis=1)
    x2 = x_pad.reshape(2 * B, HALF)

    table_bf = emb_table.astype(jnp.bfloat16)
    p = _pool(x2, table_bf)

    out = pl.pallas_call(
        _linear_body,
        out_shape=jax.ShapeDtypeStruct((B, OUT), jnp.float32),
    )(p, fc_w.T, fc_b.reshape(1, OUT))
    return out


# speedup vs baseline: 1.3403x; 1.3403x over previous
import functools

import jax
import jax.numpy as jnp
from jax import lax
from jax.experimental import pallas as pl
from jax.experimental.pallas import tpu as pltpu
from jax.experimental.pallas import tpu_sc as plsc

B = 4096
L = 200
LPAD = 208
HALF = LPAD // 2
E = 64
EP = 128
OUT = 10

NC = 2
NS = 16
NW = NC * NS
ROWS_PER_W = B // NW

NB = 4
PH = 2
RPP = ROWS_PER_W // PH


def _pool_body(x2_hbm, table_hbm, out_hbm, idx_v, rows_v, p_buf, *sems):
    wid = lax.axis_index("s") * NC + lax.axis_index("c")
    base = wid * ROWS_PER_W

    neg = jnp.full((16,), -jnp.inf, dtype=jnp.float32)

    def issue(slot, i):
        for h in range(2):
            pltpu.async_copy(
                table_hbm.at[idx_v.at[2 * i + h]],
                rows_v.at[pl.ds((2 * slot + h) * HALF, HALF)], sems[slot])

    def drain(slot):
        for h in range(2):
            pltpu.make_async_copy(
                table_hbm.at[idx_v.at[h]],
                rows_v.at[pl.ds((2 * slot + h) * HALF, HALF)],
                sems[slot]).wait()

    for ph in range(PH):
        pltpu.sync_copy(
            x2_hbm.at[pl.ds(base * 2 + ph * 2 * RPP, 2 * RPP)], idx_v)

        for s in range(NB):
            issue(s, s)

        def blk_body(g, carry):
            for s in range(NB):
                i = g * NB + s
                drain(s)

                def seq_body(j, acc):
                    accs = list(acc)
                    for jj in range(8):
                        r = rows_v.at[2 * s * HALF + j * 8 + jj]
                        for v in range(4):
                            accs[v] = jnp.maximum(
                                accs[v], r[pl.ds(v * 16, 16)])
                    return tuple(accs)

                acc = lax.fori_loop(0, 2 * HALF // 8, seq_body,
                                    (neg, neg, neg, neg))
                for v in range(4):
                    p_buf[ph * RPP + i, pl.ds(v * 16, 16)] = acc[v]

                nxt = i + NB

                @pl.when(nxt < RPP)
                def _():
                    issue(s, nxt)
            return carry

        lax.fori_loop(0, RPP // NB, blk_body, 0)

    pltpu.sync_copy(p_buf, out_hbm.at[pl.ds(base, ROWS_PER_W)])


_pool = functools.partial(
    pl.kernel,
    mesh=plsc.VectorSubcoreMesh(
        core_axis_name="c", subcore_axis_name="s",
        num_cores=NC, num_subcores=NS,
    ),
    out_type=jax.ShapeDtypeStruct((B, E), jnp.float32),
    scratch_types=[
        pltpu.VMEM((2 * RPP, HALF), jnp.int32),
        pltpu.VMEM((2 * NB * HALF, EP), jnp.float32),
        pltpu.VMEM((ROWS_PER_W, E), jnp.float32),
    ] + [pltpu.SemaphoreType.DMA] * NB,
    compiler_params=pltpu.CompilerParams(use_tc_tiling_on_sc=False),
)(_pool_body)


VB = 2048


def _repack_body(t_ref, o_ref):
    bt = t_ref[...].T
    o_ref[...] = jnp.concatenate(
        [bt, jnp.zeros((VB, EP - E), jnp.float32)], axis=1)


def _repack(table_t):
    grid = (1000000 + VB - 1) // VB
    return pl.pallas_call(
        _repack_body,
        grid=(grid,),
        in_specs=[pl.BlockSpec((E, VB), lambda k: (0, k))],
        out_specs=pl.BlockSpec((VB, EP), lambda k: (k, 0)),
        out_shape=jax.ShapeDtypeStruct((1000000, EP), jnp.float32),
    )(table_t)


def _linear_body(p_ref, w_ref, b_ref, o_ref):
    h = jnp.maximum(p_ref[...], 0.0)
    o_ref[...] = (
        jnp.dot(h, w_ref[...], preferred_element_type=jnp.float32) + b_ref[...]
    )


def kernel(x, emb_table, fc_w, fc_b):
    x = x.astype(jnp.int32)
    x_pad = jnp.concatenate([x, x[:, L - (LPAD - L):]], axis=1)
    x2 = x_pad.reshape(2 * B, HALF)

    table_pad = _repack(emb_table.T)
    p = _pool(x2, table_pad)

    out = pl.pallas_call(
        _linear_body,
        out_shape=jax.ShapeDtypeStruct((B, OUT), jnp.float32),
    )(p, fc_w.T, fc_b.reshape(1, OUT))
    return out


# speedup vs baseline: 1.6620x; 1.2401x over previous
import functools

import jax
import jax.numpy as jnp
from jax import lax
from jax.experimental import pallas as pl
from jax.experimental.pallas import tpu as pltpu
from jax.experimental.pallas import tpu_sc as plsc

B = 4096
L = 200
LPAD = 208
HALF = LPAD // 2
E = 64
EP = 128
OUT = 10

NC = 2
NS = 16
NW = NC * NS
ROWS_PER_W = B // NW

NB = 4
PH = 2
RPP = ROWS_PER_W // PH


def _pool_body(x2_hbm, table_hbm, out_hbm, idx_v, rows_v, p_buf, *sems):
    wid = lax.axis_index("s") * NC + lax.axis_index("c")
    base = wid * ROWS_PER_W

    neg = jnp.full((16,), -jnp.inf, dtype=jnp.float32)

    def issue(slot, i):
        for h in range(2):
            pltpu.async_copy(
                table_hbm.at[idx_v.at[2 * i + h]],
                rows_v.at[pl.ds((2 * slot + h) * HALF, HALF)], sems[slot])

    def drain(slot):
        for h in range(2):
            pltpu.make_async_copy(
                table_hbm.at[idx_v.at[h]],
                rows_v.at[pl.ds((2 * slot + h) * HALF, HALF)],
                sems[slot]).wait()

    for ph in range(PH):
        pltpu.sync_copy(
            x2_hbm.at[pl.ds(base * 2 + ph * 2 * RPP, 2 * RPP)], idx_v)

        for s in range(NB):
            issue(s, s)

        def blk_body(g, carry):
            for s in range(NB):
                i = g * NB + s
                drain(s)

                def seq_body(j, acc):
                    accs = list(acc)
                    for jj in range(8):
                        r = rows_v.at[2 * s * HALF + j * 8 + jj]
                        for v in range(4):
                            accs[v] = jnp.maximum(
                                accs[v], r[pl.ds(v * 16, 16)])
                    return tuple(accs)

                acc = lax.fori_loop(0, 2 * HALF // 8, seq_body,
                                    (neg, neg, neg, neg))
                for v in range(4):
                    p_buf[ph * RPP + i, pl.ds(v * 16, 16)] = acc[v]

                nxt = i + NB

                @pl.when(nxt < RPP)
                def _():
                    issue(s, nxt)
            return carry

        lax.fori_loop(0, RPP // NB, blk_body, 0)

    pltpu.sync_copy(p_buf, out_hbm.at[pl.ds(base, ROWS_PER_W)])


_pool = functools.partial(
    pl.kernel,
    mesh=plsc.VectorSubcoreMesh(
        core_axis_name="c", subcore_axis_name="s",
        num_cores=NC, num_subcores=NS,
    ),
    out_type=jax.ShapeDtypeStruct((B, E), jnp.float32),
    scratch_types=[
        pltpu.VMEM((2 * RPP, HALF), jnp.int32),
        pltpu.VMEM((2 * NB * HALF, EP), jnp.float32),
        pltpu.VMEM((ROWS_PER_W, E), jnp.float32),
    ] + [pltpu.SemaphoreType.DMA] * NB,
    compiler_params=pltpu.CompilerParams(use_tc_tiling_on_sc=False),
)(_pool_body)


VB = 4096


def _repack_body(t_ref, o_ref):
    bt = t_ref[...].T
    o_ref[...] = jnp.concatenate(
        [bt, jnp.zeros((VB, EP - E), jnp.float32)], axis=1)


def _repack(table_t):
    grid = (1000000 + VB - 1) // VB
    return pl.pallas_call(
        _repack_body,
        grid=(grid,),
        in_specs=[pl.BlockSpec((E, VB), lambda k: (0, k))],
        out_specs=pl.BlockSpec((VB, EP), lambda k: (k, 0)),
        out_shape=jax.ShapeDtypeStruct((1000000, EP), jnp.float32),
    )(table_t)


def _linear_body(p_ref, w_ref, b_ref, o_ref):
    h = jnp.maximum(p_ref[...], 0.0)
    o_ref[...] = (
        jnp.dot(h, w_ref[...], preferred_element_type=jnp.float32) + b_ref[...]
    )


def kernel(x, emb_table, fc_w, fc_b):
    x = x.astype(jnp.int32)
    x_pad = jnp.concatenate([x, x[:, L - (LPAD - L):]], axis=1)
    x2 = x_pad.reshape(2 * B, HALF)

    table_pad = _repack(emb_table.T)
    p = _pool(x2, table_pad)

    out = pl.pallas_call(
        _linear_body,
        out_shape=jax.ShapeDtypeStruct((B, OUT), jnp.float32),
    )(p, fc_w.T, fc_b.reshape(1, OUT))
    return out


# speedup vs baseline: 1.9130x; 1.1510x over previous
import functools

import jax
import jax.numpy as jnp
from jax import lax
from jax.experimental import pallas as pl
from jax.experimental.pallas import tpu as pltpu
from jax.experimental.pallas import tpu_sc as plsc

B = 4096
L = 200
LPAD = 208
HALF = LPAD // 2
E = 64
EP = 128
OUT = 10

NC = 2
NS = 16
NW = NC * NS
ROWS_PER_W = B // NW

NB = 4
PH = 2
RPP = ROWS_PER_W // PH


def _pool_body(x2_hbm, table_hbm, out_hbm, idx_v, rows_v, p_buf, *sems):
    wid = lax.axis_index("s") * NC + lax.axis_index("c")
    base = wid * ROWS_PER_W

    neg = jnp.full((16,), -jnp.inf, dtype=jnp.float32)

    def issue(slot, i):
        for h in range(2):
            pltpu.async_copy(
                table_hbm.at[idx_v.at[2 * i + h]],
                rows_v.at[pl.ds((2 * slot + h) * HALF, HALF)], sems[slot])

    def drain(slot):
        for h in range(2):
            pltpu.make_async_copy(
                table_hbm.at[idx_v.at[h]],
                rows_v.at[pl.ds((2 * slot + h) * HALF, HALF)],
                sems[slot]).wait()

    for ph in range(PH):
        pltpu.sync_copy(
            x2_hbm.at[pl.ds(base * 2 + ph * 2 * RPP, 2 * RPP)], idx_v)

        for s in range(NB):
            issue(s, s)

        def blk_body(g, carry):
            for s in range(NB):
                i = g * NB + s
                drain(s)

                def seq_body(j, acc):
                    accs = list(acc)
                    for jj in range(8):
                        r = rows_v.at[2 * s * HALF + j * 8 + jj]
                        for v in range(4):
                            accs[v] = jnp.maximum(
                                accs[v], r[pl.ds(v * 16, 16)])
                    return tuple(accs)

                acc = lax.fori_loop(0, 2 * HALF // 8, seq_body,
                                    (neg, neg, neg, neg))
                for v in range(4):
                    p_buf[ph * RPP + i, pl.ds(v * 16, 16)] = acc[v]

                nxt = i + NB

                @pl.when(nxt < RPP)
                def _():
                    issue(s, nxt)
            return carry

        lax.fori_loop(0, RPP // NB, blk_body, 0)

    pltpu.sync_copy(p_buf, out_hbm.at[pl.ds(base, ROWS_PER_W)])


_pool = functools.partial(
    pl.kernel,
    mesh=plsc.VectorSubcoreMesh(
        core_axis_name="c", subcore_axis_name="s",
        num_cores=NC, num_subcores=NS,
    ),
    out_type=jax.ShapeDtypeStruct((B, E), jnp.float32),
    scratch_types=[
        pltpu.VMEM((2 * RPP, HALF), jnp.int32),
        pltpu.VMEM((2 * NB * HALF, EP), jnp.float32),
        pltpu.VMEM((ROWS_PER_W, E), jnp.float32),
    ] + [pltpu.SemaphoreType.DMA] * NB,
    compiler_params=pltpu.CompilerParams(use_tc_tiling_on_sc=False),
)(_pool_body)


VB = 8192


def _repack_body(t_ref, o_ref):
    bt = t_ref[...].T
    o_ref[...] = jnp.concatenate(
        [bt, jnp.zeros((VB, EP - E), jnp.float32)], axis=1)


def _repack(table_t):
    grid = (1000000 + VB - 1) // VB
    return pl.pallas_call(
        _repack_body,
        grid=(grid,),
        in_specs=[pl.BlockSpec((E, VB), lambda k: (0, k))],
        out_specs=pl.BlockSpec((VB, EP), lambda k: (k, 0)),
        out_shape=jax.ShapeDtypeStruct((1000000, EP), jnp.float32),
    )(table_t)


def _linear_body(p_ref, w_ref, b_ref, o_ref):
    h = jnp.maximum(p_ref[...], 0.0)
    o_ref[...] = (
        jnp.dot(h, w_ref[...], preferred_element_type=jnp.float32) + b_ref[...]
    )


def kernel(x, emb_table, fc_w, fc_b):
    x = x.astype(jnp.int32)
    x_pad = jnp.concatenate([x, x[:, L - (LPAD - L):]], axis=1)
    x2 = x_pad.reshape(2 * B, HALF)

    table_pad = _repack(emb_table.T)
    p = _pool(x2, table_pad)

    out = pl.pallas_call(
        _linear_body,
        out_shape=jax.ShapeDtypeStruct((B, OUT), jnp.float32),
    )(p, fc_w.T, fc_b.reshape(1, OUT))
    return out


# speedup vs baseline: 1.9893x; 1.0399x over previous
import functools

import jax
import jax.numpy as jnp
from jax import lax
from jax.experimental import pallas as pl
from jax.experimental.pallas import tpu as pltpu
from jax.experimental.pallas import tpu_sc as plsc

B = 4096
L = 200
LPAD = 208
HALF = LPAD // 2
E = 64
EP = 128
OUT = 10

NC = 2
NS = 16
NW = NC * NS
ROWS_PER_W = B // NW

NB = 4
PH = 2
RPP = ROWS_PER_W // PH


def _pool_body(x2_hbm, table_hbm, out_hbm, idx_v, rows_v, p_buf, *sems):
    wid = lax.axis_index("s") * NC + lax.axis_index("c")
    base = wid * ROWS_PER_W

    neg = jnp.full((16,), -jnp.inf, dtype=jnp.float32)

    def issue(slot, i):
        for h in range(2):
            pltpu.async_copy(
                table_hbm.at[idx_v.at[2 * i + h]],
                rows_v.at[pl.ds((2 * slot + h) * HALF, HALF)], sems[slot])

    def drain(slot):
        for h in range(2):
            pltpu.make_async_copy(
                table_hbm.at[idx_v.at[h]],
                rows_v.at[pl.ds((2 * slot + h) * HALF, HALF)],
                sems[slot]).wait()

    for ph in range(PH):
        pltpu.sync_copy(
            x2_hbm.at[pl.ds(base * 2 + ph * 2 * RPP, 2 * RPP)], idx_v)

        for s in range(NB):
            issue(s, s)

        def blk_body(g, carry):
            for s in range(NB):
                i = g * NB + s
                drain(s)

                def seq_body(j, acc):
                    accs = list(acc)
                    for jj in range(8):
                        r = rows_v.at[2 * s * HALF + j * 8 + jj]
                        for v in range(4):
                            accs[v] = jnp.maximum(
                                accs[v], r[pl.ds(v * 16, 16)])
                    return tuple(accs)

                acc = lax.fori_loop(0, 2 * HALF // 8, seq_body,
                                    (neg, neg, neg, neg))
                for v in range(4):
                    p_buf[ph * RPP + i, pl.ds(v * 16, 16)] = acc[v]

                nxt = i + NB

                @pl.when(nxt < RPP)
                def _():
                    issue(s, nxt)
            return carry

        lax.fori_loop(0, RPP // NB, blk_body, 0)

    pltpu.sync_copy(p_buf, out_hbm.at[pl.ds(base, ROWS_PER_W)])


_pool = functools.partial(
    pl.kernel,
    mesh=plsc.VectorSubcoreMesh(
        core_axis_name="c", subcore_axis_name="s",
        num_cores=NC, num_subcores=NS,
    ),
    out_type=jax.ShapeDtypeStruct((B, E), jnp.float32),
    scratch_types=[
        pltpu.VMEM((2 * RPP, HALF), jnp.int32),
        pltpu.VMEM((2 * NB * HALF, EP), jnp.float32),
        pltpu.VMEM((ROWS_PER_W, E), jnp.float32),
    ] + [pltpu.SemaphoreType.DMA] * NB,
    compiler_params=pltpu.CompilerParams(use_tc_tiling_on_sc=False),
)(_pool_body)


VB = 16384


def _repack_body(t_ref, o_ref):
    bt = t_ref[...].T
    o_ref[...] = jnp.concatenate(
        [bt, jnp.zeros((VB, EP - E), jnp.float32)], axis=1)


def _repack(table_t):
    grid = (1000000 + VB - 1) // VB
    return pl.pallas_call(
        _repack_body,
        grid=(grid,),
        in_specs=[pl.BlockSpec((E, VB), lambda k: (0, k))],
        out_specs=pl.BlockSpec((VB, EP), lambda k: (k, 0)),
        out_shape=jax.ShapeDtypeStruct((1000000, EP), jnp.float32),
    )(table_t)


def _linear_body(p_ref, w_ref, b_ref, o_ref):
    h = jnp.maximum(p_ref[...], 0.0)
    o_ref[...] = (
        jnp.dot(h, w_ref[...], preferred_element_type=jnp.float32) + b_ref[...]
    )


def kernel(x, emb_table, fc_w, fc_b):
    x = x.astype(jnp.int32)
    x_pad = jnp.concatenate([x, x[:, L - (LPAD - L):]], axis=1)
    x2 = x_pad.reshape(2 * B, HALF)

    table_pad = _repack(emb_table.T)
    p = _pool(x2, table_pad)

    out = pl.pallas_call(
        _linear_body,
        out_shape=jax.ShapeDtypeStruct((B, OUT), jnp.float32),
    )(p, fc_w.T, fc_b.reshape(1, OUT))
    return out


# speedup vs baseline: 2.0200x; 1.0155x over previous
import functools

import jax
import jax.numpy as jnp
from jax import lax
from jax.experimental import pallas as pl
from jax.experimental.pallas import tpu as pltpu
from jax.experimental.pallas import tpu_sc as plsc

B = 4096
L = 200
LPAD = 208
HALF = LPAD // 2
E = 64
EP = 128
OUT = 10

NC = 2
NS = 16
NW = NC * NS
ROWS_PER_W = B // NW

NB = 4
PH = 2
RPP = ROWS_PER_W // PH


def _pool_body(x2_hbm, table_hbm, out_hbm, idx_v, rows_v, p_buf, *sems):
    wid = lax.axis_index("s") * NC + lax.axis_index("c")
    base = wid * ROWS_PER_W

    neg = jnp.full((16,), -jnp.inf, dtype=jnp.float32)

    def issue(slot, i):
        for h in range(2):
            pltpu.async_copy(
                table_hbm.at[idx_v.at[2 * i + h]],
                rows_v.at[pl.ds((2 * slot + h) * HALF, HALF)], sems[slot])

    def drain(slot):
        for h in range(2):
            pltpu.make_async_copy(
                table_hbm.at[idx_v.at[h]],
                rows_v.at[pl.ds((2 * slot + h) * HALF, HALF)],
                sems[slot]).wait()

    for ph in range(PH):
        pltpu.sync_copy(
            x2_hbm.at[pl.ds(base * 2 + ph * 2 * RPP, 2 * RPP)], idx_v)

        for s in range(NB):
            issue(s, s)

        def blk_body(g, carry):
            for s in range(NB):
                i = g * NB + s
                drain(s)

                def seq_body(j, acc):
                    accs = list(acc)
                    for jj in range(8):
                        r = rows_v.at[2 * s * HALF + j * 8 + jj]
                        for v in range(4):
                            accs[v] = jnp.maximum(
                                accs[v], r[pl.ds(v * 16, 16)])
                    return tuple(accs)

                acc = lax.fori_loop(0, 2 * HALF // 8, seq_body,
                                    (neg, neg, neg, neg))
                for v in range(4):
                    p_buf[ph * RPP + i, pl.ds(v * 16, 16)] = acc[v]

                nxt = i + NB

                @pl.when(nxt < RPP)
                def _():
                    issue(s, nxt)
            return carry

        lax.fori_loop(0, RPP // NB, blk_body, 0)

    pltpu.sync_copy(p_buf, out_hbm.at[pl.ds(base, ROWS_PER_W)])


_pool = functools.partial(
    pl.kernel,
    mesh=plsc.VectorSubcoreMesh(
        core_axis_name="c", subcore_axis_name="s",
        num_cores=NC, num_subcores=NS,
    ),
    out_type=jax.ShapeDtypeStruct((B, E), jnp.float32),
    scratch_types=[
        pltpu.VMEM((2 * RPP, HALF), jnp.int32),
        pltpu.VMEM((2 * NB * HALF, EP), jnp.float32),
        pltpu.VMEM((ROWS_PER_W, E), jnp.float32),
    ] + [pltpu.SemaphoreType.DMA] * NB,
    compiler_params=pltpu.CompilerParams(use_tc_tiling_on_sc=False),
)(_pool_body)


VB = 32768


def _repack_body(t_ref, o_ref):
    bt = t_ref[...].T
    o_ref[...] = jnp.concatenate(
        [bt, jnp.zeros((VB, EP - E), jnp.float32)], axis=1)


def _repack(table_t):
    grid = (1000000 + VB - 1) // VB
    return pl.pallas_call(
        _repack_body,
        grid=(grid,),
        in_specs=[pl.BlockSpec((E, VB), lambda k: (0, k))],
        out_specs=pl.BlockSpec((VB, EP), lambda k: (k, 0)),
        out_shape=jax.ShapeDtypeStruct((1000000, EP), jnp.float32),
    )(table_t)


def _linear_body(p_ref, w_ref, b_ref, o_ref):
    h = jnp.maximum(p_ref[...], 0.0)
    o_ref[...] = (
        jnp.dot(h, w_ref[...], preferred_element_type=jnp.float32) + b_ref[...]
    )


def kernel(x, emb_table, fc_w, fc_b):
    x = x.astype(jnp.int32)
    x_pad = jnp.concatenate([x, x[:, L - (LPAD - L):]], axis=1)
    x2 = x_pad.reshape(2 * B, HALF)

    table_pad = _repack(emb_table.T)
    p = _pool(x2, table_pad)

    out = pl.pallas_call(
        _linear_body,
        out_shape=jax.ShapeDtypeStruct((B, OUT), jnp.float32),
    )(p, fc_w.T, fc_b.reshape(1, OUT))
    return out
